# baseline (speedup 1.0000x reference)
import functools

import jax
import jax.numpy as jnp
from jax import lax
from jax.experimental import pallas as pl
from jax.experimental.pallas import tpu as pltpu
from jax.experimental.pallas import tpu_sc as plsc

N = 10000
E = 320000
D = 128
NC = 2
NS = 16
NW = NC * NS
EPT = E // NW
CHUNK = 40
NPAD = 10240
RPT = NPAD // NS
ZR = 128

_mesh = plsc.VectorSubcoreMesh(core_axis_name="c", subcore_axis_name="s")


@functools.partial(
    pl.kernel,
    out_type=jax.ShapeDtypeStruct((NW * N,), jnp.float32),
    mesh=_mesh,
    scratch_types=[
        pltpu.VMEM((EPT,), jnp.int32),
        pltpu.VMEM((N,), jnp.float32),
    ],
    compiler_params=pltpu.CompilerParams(needs_layout_passes=False),
)
def _deg_kernel(src_hbm, out_hbm, src_v, hist_v):
    c = lax.axis_index("c")
    s = lax.axis_index("s")
    wid = s * NC + c

    pltpu.sync_copy(src_hbm.at[pl.ds(wid * EPT, EPT)], src_v)

    zeros = jnp.zeros((16,), jnp.float32)

    def zbody(i, carry):
        hist_v[pl.ds(i * 16, 16)] = zeros
        return carry

    lax.fori_loop(0, N // 16, zbody, 0, unroll=4)

    ones = jnp.ones((16,), jnp.float32)

    def body(i, carry):
        idx = src_v[pl.ds(i * 16, 16)]
        plsc.addupdate_scatter(hist_v, [idx], ones)
        return carry

    lax.fori_loop(0, EPT // 16, body, 0, unroll=4)

    pltpu.sync_copy(hist_v, out_hbm.at[pl.ds(wid * N, N)])


def _prep_body(x_ref, hist_ref, xn_ref, invd_ref):
    deg = jnp.sum(hist_ref[...], axis=0)
    invd = lax.rsqrt(deg + 1.0)
    xn_ref[...] = x_ref[...] * invd[:, None]
    invd_ref[...] = invd[:, None]


_PR = 1024

_prep_call = pl.pallas_call(
    _prep_body,
    grid=((N + _PR - 1) // _PR,),
    in_specs=[
        pl.BlockSpec((_PR, D), lambda i: (i, 0)),
        pl.BlockSpec((NW, _PR), lambda i: (0, i)),
    ],
    out_specs=[
        pl.BlockSpec((_PR, D), lambda i: (i, 0)),
        pl.BlockSpec((_PR, 1), lambda i: (i, 0)),
    ],
    out_shape=[
        jax.ShapeDtypeStruct((NPAD, D), jnp.float32),
        jax.ShapeDtypeStruct((N, 1), jnp.float32),
    ],
)


TCH = E // CHUNK
CPT = TCH // NW
NSLOT = 6


@functools.partial(
    pl.kernel,
    out_type=jax.ShapeDtypeStruct((NC, NPAD, D), jnp.float32),
    mesh=_mesh,
    scratch_types=[
        pltpu.VMEM((CPT * CHUNK,), jnp.int32),
        [pltpu.VMEM((CHUNK,), jnp.int32) for _ in range(NSLOT)],
        [pltpu.VMEM((CHUNK, D), jnp.float32) for _ in range(NSLOT)],
        pltpu.VMEM((CHUNK, D), jnp.float32),
        pltpu.VMEM_SHARED((NPAD, D), jnp.float32),
        [pltpu.SemaphoreType.DMA for _ in range(NSLOT)],
        [pltpu.SemaphoreType.DMA for _ in range(NSLOT)],
        [pltpu.SemaphoreType.DMA for _ in range(NSLOT)],
    ],
)
def _agg_kernel(xn_hbm, src_hbm, dst_hbm, out_hbm, sidx_all, didx, rows,
                zbuf, acc_sh, sem_i, sem_g, sem_s):
    c = lax.axis_index("c")
    s = lax.axis_index("s")
    wid = s * NC + c
    g0 = wid * CPT

    pltpu.sync_copy(src_hbm.at[pl.ds(g0 * CHUNK, CPT * CHUNK)], sidx_all)

    for k in range(NSLOT):
        pltpu.async_copy(dst_hbm.at[pl.ds((g0 + k) * CHUNK, CHUNK)],
                         didx[k], sem_i[k])
        pltpu.async_copy(xn_hbm.at[sidx_all.at[pl.ds(k * CHUNK, CHUNK)]],
                         rows[k], sem_g[k])

    @pl.when(c == 0)
    def _():
        pltpu.sync_copy(xn_hbm.at[pl.ds(s * RPT, RPT)],
                        acc_sh.at[pl.ds(s * RPT, RPT)])

    @pl.when(c != 0)
    def _():
        zeros = jnp.zeros((16,), jnp.float32)

        def zbody(i, carry):
            r = i // (D // 16)
            k = i % (D // 16)
            zbuf[r, pl.ds(k * 16, 16)] = zeros
            return carry

        lax.fori_loop(0, CHUNK * (D // 16), zbody, 0, unroll=4)

        def zcopy(i, carry):
            pltpu.sync_copy(zbuf,
                            acc_sh.at[pl.ds(s * RPT + i * CHUNK, CHUNK)])
            return carry

        lax.fori_loop(0, RPT // CHUNK, zcopy, 0)

    plsc.subcore_barrier()

    def body(u, carry):
        for k in range(NSLOT):
            j = u * NSLOT + k

            @pl.when(jnp.logical_and(j < CPT, j >= NSLOT))
            def _(k=k):
                pltpu.make_async_copy(rows[k], acc_sh.at[didx[k]],
                                      sem_s[k]).wait()

            @pl.when(jnp.logical_and(j < CPT, j >= NSLOT))
            def _(j=j, k=k):
                pltpu.async_copy(dst_hbm.at[pl.ds((g0 + j) * CHUNK, CHUNK)],
                                 didx[k], sem_i[k])
                pltpu.async_copy(
                    xn_hbm.at[sidx_all.at[pl.ds(j * CHUNK, CHUNK)]],
                    rows[k], sem_g[k])

        for k in range(NSLOT):
            j = u * NSLOT + k

            @pl.when(j < CPT)
            def _(k=k):
                pltpu.make_async_copy(
                    xn_hbm.at[sidx_all.at[pl.ds(0, CHUNK)]], rows[k],
                    sem_g[k]).wait()
                pltpu.make_async_copy(dst_hbm.at[pl.ds(0, CHUNK)], didx[k],
                                      sem_i[k]).wait()
                pltpu.async_copy(rows[k], acc_sh.at[didx[k]], sem_s[k],
                                 add=True)

        return carry

    lax.fori_loop(0, (CPT + NSLOT - 1) // NSLOT, body, 0)

    for k in range(NSLOT):
        pltpu.make_async_copy(rows[k], acc_sh.at[didx[k]], sem_s[k]).wait()

    plsc.subcore_barrier()

    pltpu.sync_copy(acc_sh.at[pl.ds(s * RPT, RPT)],
                    out_hbm.at[c, pl.ds(s * RPT, RPT)])


def _out_body(agg_ref, invd_ref, w_ref, b_ref, o_ref):
    z = (agg_ref[0] + agg_ref[1]) * invd_ref[...]
    o_ref[...] = lax.dot_general(
        z, w_ref[...], (((1,), (1,)), ((), ())),
        preferred_element_type=jnp.float32) + b_ref[...]


_R = 1000

_out_call = pl.pallas_call(
    _out_body,
    grid=(N // _R,),
    in_specs=[
        pl.BlockSpec((NC, _R, D), lambda i: (0, i, 0)),
        pl.BlockSpec((_R, 1), lambda i: (i, 0)),
        pl.BlockSpec((D, D), lambda i: (0, 0)),
        pl.BlockSpec((1, D), lambda i: (0, 0)),
    ],
    out_specs=pl.BlockSpec((_R, D), lambda i: (i, 0)),
    out_shape=jax.ShapeDtypeStruct((N, D), jnp.float32),
)


def kernel(x, edge_index, W, b):
    src = edge_index[0]
    dst = edge_index[1]
    hist = _deg_kernel(src).reshape(NW, N)
    xn, invd = _prep_call(x, hist)
    agg2 = _agg_kernel(xn, src, dst)
    return _out_call(agg2, invd, W, b.reshape(1, D))


# speedup vs baseline: 1.1042x; 1.1042x over previous
import functools

import jax
import jax.numpy as jnp
from jax import lax
from jax.experimental import pallas as pl
from jax.experimental.pallas import tpu as pltpu
from jax.experimental.pallas import tpu_sc as plsc

N = 10000
E = 320000
D = 128
NC = 2
NS = 16
NW = NC * NS
EPT = E // NW
CHUNK = 40
NPAD = 10240
RPT = NPAD // NS
ZR = 128

_mesh = plsc.VectorSubcoreMesh(core_axis_name="c", subcore_axis_name="s")


@functools.partial(
    pl.kernel,
    out_type=jax.ShapeDtypeStruct((NW * N,), jnp.float32),
    mesh=_mesh,
    scratch_types=[
        pltpu.VMEM((EPT,), jnp.int32),
        pltpu.VMEM((N,), jnp.float32),
    ],
    compiler_params=pltpu.CompilerParams(needs_layout_passes=False),
)
def _deg_kernel(eflat_hbm, out_hbm, src_v, hist_v):
    c = lax.axis_index("c")
    s = lax.axis_index("s")
    wid = s * NC + c

    pltpu.sync_copy(eflat_hbm.at[pl.ds(wid * EPT, EPT)], src_v)

    zeros = jnp.zeros((16,), jnp.float32)

    def zbody(i, carry):
        hist_v[pl.ds(i * 16, 16)] = zeros
        return carry

    lax.fori_loop(0, N // 16, zbody, 0, unroll=4)

    ones = jnp.ones((16,), jnp.float32)

    def body(i, carry):
        idx = src_v[pl.ds(i * 16, 16)]
        plsc.addupdate_scatter(hist_v, [idx], ones)
        return carry

    lax.fori_loop(0, EPT // 16, body, 0, unroll=4)

    pltpu.sync_copy(hist_v, out_hbm.at[pl.ds(wid * N, N)])


def _prep_body(x_ref, hist_ref, xn_ref, invd_ref):
    deg = jnp.sum(hist_ref[...], axis=0)
    invd = lax.rsqrt(deg + 1.0)
    xn_ref[pl.ds(0, N), :] = x_ref[...] * invd[:, None]
    invd_ref[...] = invd[:, None]


_prep_call = pl.pallas_call(
    _prep_body,
    out_shape=[
        jax.ShapeDtypeStruct((NPAD, D), jnp.float32),
        jax.ShapeDtypeStruct((N, 1), jnp.float32),
    ],
)


TCH = E // CHUNK
CPT = TCH // NW
NSLOT = 6


@functools.partial(
    pl.kernel,
    out_type=jax.ShapeDtypeStruct((NC, NPAD, D), jnp.float32),
    mesh=_mesh,
    scratch_types=[
        pltpu.VMEM((CPT * CHUNK,), jnp.int32),
        [pltpu.VMEM((CHUNK,), jnp.int32) for _ in range(NSLOT)],
        [pltpu.VMEM((CHUNK, D), jnp.float32) for _ in range(NSLOT)],
        pltpu.VMEM((CHUNK, D), jnp.float32),
        pltpu.VMEM_SHARED((NPAD, D), jnp.float32),
        [pltpu.SemaphoreType.DMA for _ in range(NSLOT)],
        [pltpu.SemaphoreType.DMA for _ in range(NSLOT)],
        [pltpu.SemaphoreType.DMA for _ in range(NSLOT)],
    ],
)
def _agg_kernel(xn_hbm, eflat_hbm, out_hbm, sidx_all, didx, rows,
                zbuf, acc_sh, sem_i, sem_g, sem_s):
    c = lax.axis_index("c")
    s = lax.axis_index("s")
    wid = s * NC + c
    g0 = wid * CPT

    pltpu.sync_copy(eflat_hbm.at[pl.ds(g0 * CHUNK, CPT * CHUNK)], sidx_all)

    for k in range(NSLOT):
        pltpu.async_copy(eflat_hbm.at[pl.ds(E + (g0 + k) * CHUNK, CHUNK)],
                         didx[k], sem_i[k])
        pltpu.async_copy(xn_hbm.at[sidx_all.at[pl.ds(k * CHUNK, CHUNK)]],
                         rows[k], sem_g[k])

    @pl.when(c == 0)
    def _():
        pltpu.sync_copy(xn_hbm.at[pl.ds(s * RPT, RPT)],
                        acc_sh.at[pl.ds(s * RPT, RPT)])

    @pl.when(c != 0)
    def _():
        zeros = jnp.zeros((16,), jnp.float32)

        def zbody(i, carry):
            r = i // (D // 16)
            k = i % (D // 16)
            zbuf[r, pl.ds(k * 16, 16)] = zeros
            return carry

        lax.fori_loop(0, CHUNK * (D // 16), zbody, 0, unroll=4)

        def zcopy(i, carry):
            pltpu.sync_copy(zbuf,
                            acc_sh.at[pl.ds(s * RPT + i * CHUNK, CHUNK)])
            return carry

        lax.fori_loop(0, RPT // CHUNK, zcopy, 0)

    plsc.subcore_barrier()

    def body(u, carry):
        for k in range(NSLOT):
            j = u * NSLOT + k

            @pl.when(jnp.logical_and(j < CPT, j >= NSLOT))
            def _(k=k):
                pltpu.make_async_copy(rows[k], acc_sh.at[didx[k]],
                                      sem_s[k]).wait()

            @pl.when(jnp.logical_and(j < CPT, j >= NSLOT))
            def _(j=j, k=k):
                pltpu.async_copy(
                    eflat_hbm.at[pl.ds(E + (g0 + j) * CHUNK, CHUNK)],
                    didx[k], sem_i[k])
                pltpu.async_copy(
                    xn_hbm.at[sidx_all.at[pl.ds(j * CHUNK, CHUNK)]],
                    rows[k], sem_g[k])

        for k in range(NSLOT):
            j = u * NSLOT + k

            @pl.when(j < CPT)
            def _(k=k):
                pltpu.make_async_copy(
                    xn_hbm.at[sidx_all.at[pl.ds(0, CHUNK)]], rows[k],
                    sem_g[k]).wait()
                pltpu.make_async_copy(eflat_hbm.at[pl.ds(0, CHUNK)],
                                      didx[k], sem_i[k]).wait()
                pltpu.async_copy(rows[k], acc_sh.at[didx[k]], sem_s[k],
                                 add=True)

        return carry

    lax.fori_loop(0, (CPT + NSLOT - 1) // NSLOT, body, 0)

    for k in range(NSLOT):
        pltpu.make_async_copy(rows[k], acc_sh.at[didx[k]], sem_s[k]).wait()

    plsc.subcore_barrier()

    pltpu.sync_copy(acc_sh.at[pl.ds(s * RPT, RPT)],
                    out_hbm.at[c, pl.ds(s * RPT, RPT)])


def _out_body(agg_ref, invd_ref, w_ref, b_ref, o_ref):
    z = (agg_ref[0] + agg_ref[1]) * invd_ref[...]
    o_ref[...] = lax.dot_general(
        z, w_ref[...], (((1,), (1,)), ((), ())),
        preferred_element_type=jnp.float32) + b_ref[...]


_R = 2048

_out_call = pl.pallas_call(
    _out_body,
    grid=((N + _R - 1) // _R,),
    in_specs=[
        pl.BlockSpec((NC, _R, D), lambda i: (0, i, 0)),
        pl.BlockSpec((_R, 1), lambda i: (i, 0)),
        pl.BlockSpec((D, D), lambda i: (0, 0)),
        pl.BlockSpec((1, D), lambda i: (0, 0)),
    ],
    out_specs=pl.BlockSpec((_R, D), lambda i: (i, 0)),
    out_shape=jax.ShapeDtypeStruct((N, D), jnp.float32),
)


def kernel(x, edge_index, W, b):
    eflat = edge_index.reshape(2 * E)
    hist = _deg_kernel(eflat).reshape(NW, N)
    xn, invd = _prep_call(x, hist)
    agg2 = _agg_kernel(xn, eflat)
    return _out_call(agg2, invd, W, b.reshape(1, D))
